# baseline (speedup 1.0000x reference)
import jax
import jax.numpy as jnp
from jax.experimental import pallas as pl

K_NEIGHBORS = 9
EPS = 1e-5


def _batchnorm(x, gamma, beta):
    m = jnp.mean(x, axis=0)
    v = jnp.var(x, axis=0)
    return (x - m) / jnp.sqrt(v + EPS) * gamma + beta


def _add_kernel(a_ref, b_ref, o_ref):
    o_ref[...] = a_ref[...] + b_ref[...]


def kernel(x, W1, b1, g1, be1, Wg, bg, gg, beg, W2, b2, g2, be2):
    B, N, C = x.shape
    residual = x
    h = x.reshape(B * N, C) @ W1 + b1
    h = _batchnorm(h, g1, be1)
    h = h.reshape(B, N, C)
    inner = -2.0 * jnp.einsum('bnc,bmc->bnm', h, h)
    sq = jnp.sum(h * h, axis=-1, keepdims=True)
    dist = sq + inner + jnp.swapaxes(sq, 1, 2)
    _, idx = jax.lax.top_k(-dist, K_NEIGHBORS)
    neighbors = jax.vmap(lambda xb, ib: xb[ib])(h, idx)
    diff = neighbors - h[:, :, None, :]
    agg = jnp.max(diff, axis=2)
    agg = jnp.concatenate([h, agg], axis=-1).reshape(B * N, 2 * C)
    o = agg @ Wg + bg
    o = _batchnorm(o, gg, beg)
    o = jax.nn.gelu(o, approximate=False)
    o = o @ W2 + b2
    o = _batchnorm(o, g2, be2)
    o = o.reshape(B, N, C)
    return pl.pallas_call(
        _add_kernel,
        out_shape=jax.ShapeDtypeStruct((B, N, C), x.dtype),
    )(o, residual)


# speedup vs baseline: 5.6547x; 5.6547x over previous
import jax
import jax.numpy as jnp
from jax.experimental import pallas as pl

K_NEIGHBORS = 9
EPS = 1e-5
_HI = jax.lax.Precision.HIGHEST
_NTOT = float(16 * 1024)


def _norm(h, stats, g, be):
    m = stats[0:1, :] / _NTOT
    v = stats[1:2, :] / _NTOT - m * m
    return (h - m) / jnp.sqrt(v + EPS) * g + be


def _fc1_body(x_ref, w_ref, b_ref, h_ref, s_ref):
    h = jnp.dot(x_ref[...], w_ref[...],
                preferred_element_type=jnp.float32) + b_ref[...]
    h_ref[...] = h

    @pl.when(pl.program_id(0) == 0)
    def _():
        s_ref[...] = jnp.zeros_like(s_ref)

    s_ref[0:1, :] += jnp.sum(h, axis=0, keepdims=True)
    s_ref[1:2, :] += jnp.sum(h * h, axis=0, keepdims=True)


def _knn_agg_body(h_ref, s_ref, g_ref, be_ref, agg_ref):
    hb = _norm(h_ref[0], s_ref[...], g_ref[...], be_ref[...])
    n = hb.shape[0]
    sq = jnp.sum(hb * hb, axis=1, keepdims=True)
    gram = jax.lax.dot_general(hb, hb, (((1,), (1,)), ((), ())))
    dist = sq - 2.0 * gram + sq.T
    cols = jax.lax.broadcasted_iota(jnp.int32, (n, n), 1)
    d = dist
    agg = jnp.full(hb.shape, -3e38, dtype=jnp.float32)
    for _ in range(K_NEIGHBORS):
        mv = jnp.min(d, axis=1, keepdims=True)
        am = jnp.min(jnp.where(d == mv, cols, n), axis=1, keepdims=True)
        oh = (cols == am).astype(jnp.float32)
        nb = jax.lax.dot_general(oh, hb, (((1,), (0,)), ((), ())), precision=_HI)
        agg = jnp.maximum(agg, nb)
        d = jnp.where(cols == am, 3e38, d)
    agg_ref[0] = agg - hb


def _fcg_body(h_ref, s1_ref, g1_ref, be1_ref, a_ref, wg_ref, bg_ref,
              o_ref, s_ref):
    c = h_ref.shape[1]
    h = _norm(h_ref[...], s1_ref[...], g1_ref[...], be1_ref[...])
    o = (jnp.dot(h, wg_ref[:c, :], preferred_element_type=jnp.float32,
                 precision=_HI)
         + jnp.dot(a_ref[...], wg_ref[c:, :], preferred_element_type=jnp.float32,
                   precision=_HI)
         + bg_ref[...])
    o_ref[...] = o

    @pl.when(pl.program_id(0) == 0)
    def _():
        s_ref[...] = jnp.zeros_like(s_ref)

    s_ref[0:1, :] += jnp.sum(o, axis=0, keepdims=True)
    s_ref[1:2, :] += jnp.sum(o * o, axis=0, keepdims=True)


def _fc2_body(o_ref, sg_ref, gg_ref, beg_ref, w2_ref, b2_ref, o2_ref, s_ref):
    o = _norm(o_ref[...], sg_ref[...], gg_ref[...], beg_ref[...])
    o = 0.5 * o * (1.0 + jax.lax.erf(o * 0.7071067811865476))
    o2 = jnp.dot(o, w2_ref[...], preferred_element_type=jnp.float32,
                 precision=_HI) + b2_ref[...]
    o2_ref[...] = o2

    @pl.when(pl.program_id(0) == 0)
    def _():
        s_ref[...] = jnp.zeros_like(s_ref)

    s_ref[0:1, :] += jnp.sum(o2, axis=0, keepdims=True)
    s_ref[1:2, :] += jnp.sum(o2 * o2, axis=0, keepdims=True)


def _bn_res_body(o2_ref, s2_ref, g2_ref, be2_ref, x_ref, out_ref):
    out_ref[...] = (_norm(o2_ref[...], s2_ref[...], g2_ref[...], be2_ref[...])
                    + x_ref[...])


def kernel(x, W1, b1, g1, be1, Wg, bg, gg, beg, W2, b2, g2, be2):
    B, N, C = x.shape
    R = B * N
    NB = 16
    RB = R // NB
    xf = x.reshape(R, C)
    row = lambda v: v.reshape(1, -1)

    rows_spec = pl.BlockSpec((RB, C), lambda i: (i, 0))
    stat_spec = pl.BlockSpec((2, C), lambda i: (0, 0))
    vec_spec = pl.BlockSpec((1, C), lambda i: (0, 0))
    full = lambda a: pl.BlockSpec(a.shape, lambda i: tuple(0 for _ in a.shape))

    hraw, s1 = pl.pallas_call(
        _fc1_body,
        grid=(NB,),
        in_specs=[rows_spec, full(W1), vec_spec],
        out_specs=[rows_spec, stat_spec],
        out_shape=[jax.ShapeDtypeStruct((R, C), jnp.float32),
                   jax.ShapeDtypeStruct((2, C), jnp.float32)],
    )(xf, W1, row(b1))

    agg = pl.pallas_call(
        _knn_agg_body,
        grid=(B,),
        in_specs=[pl.BlockSpec((1, N, C), lambda b: (b, 0, 0)),
                  stat_spec, vec_spec, vec_spec],
        out_specs=pl.BlockSpec((1, N, C), lambda b: (b, 0, 0)),
        out_shape=jax.ShapeDtypeStruct((B, N, C), jnp.float32),
    )(hraw.reshape(B, N, C), s1, row(g1), row(be1))

    o1, sg = pl.pallas_call(
        _fcg_body,
        grid=(NB,),
        in_specs=[rows_spec, stat_spec, vec_spec, vec_spec, rows_spec,
                  full(Wg), vec_spec],
        out_specs=[rows_spec, stat_spec],
        out_shape=[jax.ShapeDtypeStruct((R, C), jnp.float32),
                   jax.ShapeDtypeStruct((2, C), jnp.float32)],
    )(hraw, s1, row(g1), row(be1), agg.reshape(R, C), Wg, row(bg))

    o2, s2 = pl.pallas_call(
        _fc2_body,
        grid=(NB,),
        in_specs=[rows_spec, stat_spec, vec_spec, vec_spec, full(W2), vec_spec],
        out_specs=[rows_spec, stat_spec],
        out_shape=[jax.ShapeDtypeStruct((R, C), jnp.float32),
                   jax.ShapeDtypeStruct((2, C), jnp.float32)],
    )(o1, sg, row(gg), row(beg), W2, row(b2))

    out = pl.pallas_call(
        _bn_res_body,
        grid=(NB,),
        in_specs=[rows_spec, stat_spec, vec_spec, vec_spec, rows_spec],
        out_specs=rows_spec,
        out_shape=jax.ShapeDtypeStruct((R, C), jnp.float32),
    )(o2, s2, row(g2), row(be2), xf)
    return out.reshape(B, N, C)


# speedup vs baseline: 13.7354x; 2.4290x over previous
import functools

import jax
import jax.numpy as jnp
from jax import lax
from jax.experimental import pallas as pl
from jax.experimental.pallas import tpu as pltpu
from jax.experimental.pallas import tpu_sc as plsc

K_NEIGHBORS = 9
EPS = 1e-5
_HI = jax.lax.Precision.HIGHEST
_NTOT = float(16 * 1024)

_NC, _NS, _L = 2, 16, 16
_NW = _NC * _NS


def _norm(h, stats, g, be):
    m = stats[0:1, :] / _NTOT
    v = stats[1:2, :] / _NTOT - m * m
    return (h - m) / jnp.sqrt(v + EPS) * g + be


def _fc1_body(x_ref, w_ref, b_ref, h_ref, s_ref):
    h = jnp.dot(x_ref[...], w_ref[...],
                preferred_element_type=jnp.float32) + b_ref[...]
    h_ref[...] = h

    @pl.when(pl.program_id(0) == 0)
    def _():
        s_ref[...] = jnp.zeros_like(s_ref)

    s_ref[0:1, :] += jnp.sum(h, axis=0, keepdims=True)
    s_ref[1:2, :] += jnp.sum(h * h, axis=0, keepdims=True)


def _knn_idx_body(h_ref, s_ref, g_ref, be_ref, hn_ref, idx_ref):
    hb = _norm(h_ref[0], s_ref[...], g_ref[...], be_ref[...])
    hn_ref[0] = jnp.concatenate(
        [hb, jnp.zeros((hb.shape[0], 128 - hb.shape[1]), jnp.float32)], axis=1)
    n = hb.shape[0]
    sq = jnp.sum(hb * hb, axis=1, keepdims=True)
    gram = jax.lax.dot_general(hb, hb, (((1,), (1,)), ((), ())))
    dist = sq - 2.0 * gram + sq.T
    cols = jax.lax.broadcasted_iota(jnp.int32, (n, n), 1)
    d = dist
    ams = []
    for _ in range(K_NEIGHBORS):
        mv = jnp.min(d, axis=1, keepdims=True)
        am = jnp.min(jnp.where(d == mv, cols, n), axis=1, keepdims=True)
        ams.append(am)
        d = jnp.where(cols == am, 3e38, d)
    idx_ref[0] = jnp.concatenate(ams, axis=1) + pl.program_id(0) * n


def _make_sc_agg(R, C):
    npw = R // _NW
    ch_nodes = 8
    nch = npw // ch_nodes
    ch_idx = ch_nodes * K_NEIGHBORS
    cl = C // _L
    mesh = plsc.VectorSubcoreMesh(core_axis_name="c", subcore_axis_name="s")

    @functools.partial(
        pl.kernel,
        out_type=jax.ShapeDtypeStruct((R, 128), jnp.float32),
        mesh=mesh,
        scratch_types=[
            pltpu.VMEM((nch, ch_idx), jnp.int32),
            pltpu.VMEM((2, ch_idx, 128), jnp.float32),
            pltpu.VMEM((2, ch_nodes, 128), jnp.float32),
            pltpu.VMEM((ch_nodes, 128), jnp.float32),
            pltpu.SemaphoreType.DMA,
            pltpu.SemaphoreType.DMA,
        ],
    )
    def sc_agg(h_hbm, idx_hbm, out_hbm, idx_v, rows_v, own_v, o_v,
               sem_g, sem_o):
        wid = lax.axis_index("s") * _NC + lax.axis_index("c")
        base = wid * npw
        pltpu.sync_copy(idx_hbm.at[wid], idx_v)
        zeros = jnp.zeros((_L,), jnp.float32)
        for i in range(ch_nodes):
            for c in range(cl, 128 // _L):
                o_v[i, pl.ds(c * _L, _L)] = zeros

        def issue(ch, buf):
            pltpu.async_copy(h_hbm.at[idx_v.at[ch]], rows_v.at[buf], sem_g)
            pltpu.async_copy(h_hbm.at[pl.ds(base + ch * ch_nodes, ch_nodes)],
                             own_v.at[buf], sem_o)

        def wait(ch, buf):
            pltpu.make_async_copy(h_hbm.at[idx_v.at[ch]], rows_v.at[buf],
                                  sem_g).wait()
            pltpu.make_async_copy(
                h_hbm.at[pl.ds(base + ch * ch_nodes, ch_nodes)],
                own_v.at[buf], sem_o).wait()

        def compute(ch, buf):
            rv = rows_v.at[buf]
            ov = own_v.at[buf]
            for i in range(ch_nodes):
                for c in range(cl):
                    s = pl.ds(c * _L, _L)
                    acc = rv[i * K_NEIGHBORS, s]
                    for k in range(1, K_NEIGHBORS):
                        acc = jnp.maximum(acc, rv[i * K_NEIGHBORS + k, s])
                    o_v[i, s] = acc - ov[i, s]
            pltpu.sync_copy(o_v,
                            out_hbm.at[pl.ds(base + ch * ch_nodes, ch_nodes)])

        issue(0, 0)

        @pl.loop(0, nch, step=2)
        def _(ch2):
            for b2 in range(2):
                cur = ch2 + b2

                @pl.when(cur + 1 < nch)
                def _():
                    issue(cur + 1, 1 - b2)

                wait(cur, b2)
                compute(cur, b2)

    return sc_agg


def _fcg_body(h_ref, a_ref, wg_ref, bg_ref, o_ref, s_ref):
    c = wg_ref.shape[1]
    o = (jnp.dot(h_ref[:, :c], wg_ref[:c, :], preferred_element_type=jnp.float32,
                 precision=_HI)
         + jnp.dot(a_ref[:, :c], wg_ref[c:, :], preferred_element_type=jnp.float32,
                   precision=_HI)
         + bg_ref[...])
    o_ref[...] = o

    @pl.when(pl.program_id(0) == 0)
    def _():
        s_ref[...] = jnp.zeros_like(s_ref)

    s_ref[0:1, :] += jnp.sum(o, axis=0, keepdims=True)
    s_ref[1:2, :] += jnp.sum(o * o, axis=0, keepdims=True)


def _fc2_body(o_ref, sg_ref, gg_ref, beg_ref, w2_ref, b2_ref, o2_ref, s_ref):
    o = _norm(o_ref[...], sg_ref[...], gg_ref[...], beg_ref[...])
    o = 0.5 * o * (1.0 + jax.lax.erf(o * 0.7071067811865476))
    o2 = jnp.dot(o, w2_ref[...], preferred_element_type=jnp.float32,
                 precision=_HI) + b2_ref[...]
    o2_ref[...] = o2

    @pl.when(pl.program_id(0) == 0)
    def _():
        s_ref[...] = jnp.zeros_like(s_ref)

    s_ref[0:1, :] += jnp.sum(o2, axis=0, keepdims=True)
    s_ref[1:2, :] += jnp.sum(o2 * o2, axis=0, keepdims=True)


def _bn_res_body(o2_ref, s2_ref, g2_ref, be2_ref, x_ref, out_ref):
    out_ref[...] = (_norm(o2_ref[...], s2_ref[...], g2_ref[...], be2_ref[...])
                    + x_ref[...])


def kernel(x, W1, b1, g1, be1, Wg, bg, gg, beg, W2, b2, g2, be2):
    B, N, C = x.shape
    R = B * N
    NB = 16
    RB = R // NB
    xf = x.reshape(R, C)
    row = lambda v: v.reshape(1, -1)

    rows_spec = pl.BlockSpec((RB, C), lambda i: (i, 0))
    stat_spec = pl.BlockSpec((2, C), lambda i: (0, 0))
    vec_spec = pl.BlockSpec((1, C), lambda i: (0, 0))
    full = lambda a: pl.BlockSpec(a.shape, lambda i: tuple(0 for _ in a.shape))

    hraw, s1 = pl.pallas_call(
        _fc1_body,
        grid=(NB,),
        in_specs=[rows_spec, full(W1), vec_spec],
        out_specs=[rows_spec, stat_spec],
        out_shape=[jax.ShapeDtypeStruct((R, C), jnp.float32),
                   jax.ShapeDtypeStruct((2, C), jnp.float32)],
    )(xf, W1, row(b1))

    hn, idx = pl.pallas_call(
        _knn_idx_body,
        grid=(B,),
        in_specs=[pl.BlockSpec((1, N, C), lambda b: (b, 0, 0)),
                  stat_spec, vec_spec, vec_spec],
        out_specs=[pl.BlockSpec((1, N, 128), lambda b: (b, 0, 0)),
                   pl.BlockSpec((1, N, K_NEIGHBORS), lambda b: (b, 0, 0))],
        out_shape=[jax.ShapeDtypeStruct((B, N, 128), jnp.float32),
                   jax.ShapeDtypeStruct((B, N, K_NEIGHBORS), jnp.int32)],
    )(hraw.reshape(B, N, C), s1, row(g1), row(be1))

    hn2 = hn.reshape(R, 128)
    idx_w = idx.reshape(_NW, (R // _NW) // 8, 8 * K_NEIGHBORS)
    agg = _make_sc_agg(R, C)(hn2, idx_w)

    wide_spec = pl.BlockSpec((RB, 128), lambda i: (i, 0))
    o1, sg = pl.pallas_call(
        _fcg_body,
        grid=(NB,),
        in_specs=[wide_spec, wide_spec, full(Wg), vec_spec],
        out_specs=[rows_spec, stat_spec],
        out_shape=[jax.ShapeDtypeStruct((R, C), jnp.float32),
                   jax.ShapeDtypeStruct((2, C), jnp.float32)],
    )(hn2, agg, Wg, row(bg))

    o2, s2 = pl.pallas_call(
        _fc2_body,
        grid=(NB,),
        in_specs=[rows_spec, stat_spec, vec_spec, vec_spec, full(W2), vec_spec],
        out_specs=[rows_spec, stat_spec],
        out_shape=[jax.ShapeDtypeStruct((R, C), jnp.float32),
                   jax.ShapeDtypeStruct((2, C), jnp.float32)],
    )(o1, sg, row(gg), row(beg), W2, row(b2))

    out = pl.pallas_call(
        _bn_res_body,
        grid=(NB,),
        in_specs=[rows_spec, stat_spec, vec_spec, vec_spec, rows_spec],
        out_specs=rows_spec,
        out_shape=jax.ShapeDtypeStruct((R, C), jnp.float32),
    )(o2, s2, row(g2), row(be2), xf)
    return out.reshape(B, N, C)
